# baseline (speedup 1.0000x reference)
import jax
import jax.numpy as jnp
from jax.experimental import pallas as pl
from jax.experimental.pallas import tpu as pltpu

_B, _H, _W, _D = 64, 24, 24, 768
_K = 9
_BB = 4


def _conv_head_body(x1_ref, x2_ref, w_ref, t_ref, b_ref, o1_ref, o2_ref):
    _conv_head_one(x1_ref, w_ref, t_ref, b_ref, o1_ref)
    _conv_head_one(x2_ref, w_ref, t_ref, b_ref, o2_ref)


def _conv_head_one(x_ref, w_ref, t_ref, b_ref, o_ref):
    x = x_ref[...].reshape(_BB * _H * _W, _D)
    z = jnp.dot(x, w_ref[...], preferred_element_type=jnp.float32)
    z = z.reshape(_BB, _H, _W, 128)
    wio = jax.lax.broadcasted_iota(jnp.int32, (_BB, _H, _W, 128), 2)
    pm = jnp.where(wio == 0, 0.0, pltpu.roll(z, 1, axis=2))
    pp = jnp.where(wio == _W - 1, 0.0, pltpu.roll(z, _W - 1, axis=2))
    zrow = jnp.zeros((_BB, 1, _W, 128), jnp.float32)
    pj = [jnp.concatenate([zrow, p, zrow], axis=1) for p in (pm, z, pp)]
    terms = []
    for t in range(9):
        i, j = divmod(t, 3)
        terms.append(jax.lax.slice(
            pj[j], (0, i, 0, 0), (_BB, i + _H, _W, 128)))
    cio = jax.lax.broadcasted_iota(jnp.int32, (_BB, _H, _W, 128), 3)
    s = terms[8]
    for t in range(7, -1, -1):
        s = jnp.where(cio < _K * (t + 1), terms[t], s)
    s = s.reshape(_BB * _H * _W, 128)
    acc = jnp.dot(s, t_ref[...], preferred_element_type=jnp.float32)
    logits = jax.lax.slice(acc, (0, 0), (_BB * _H * _W, _K)) + b_ref[0, 0]
    m = jnp.max(logits, axis=-1, keepdims=True)
    e = jnp.exp(logits - m)
    den = jnp.sum(e, axis=-1, keepdims=True)
    o_ref[...] = (e / den).reshape(_BB, _H, _W, _K)


def kernel(tok2d, dw_w, pw_w, pw_b):
    dwt = dw_w.reshape(_D, 9)
    pwm = jnp.transpose(pw_w.reshape(_K, _D))
    wall = (dwt[:, :, None] * pwm[:, None, :]).reshape(_D, 81)
    wall = jnp.pad(wall, ((0, 0), (0, 47)))
    rows = jnp.arange(81)
    tmat = jnp.zeros((128, 128), jnp.float32).at[rows, rows % _K].set(1.0)
    bias = jnp.zeros((1, 1, 1, _K), jnp.float32).at[0, 0, 0, :].set(pw_b)
    out1, out2 = pl.pallas_call(
        _conv_head_body,
        grid=(_B // (2 * _BB),),
        in_specs=[
            pl.BlockSpec((_BB, _H, _W, _D), lambda b: (2 * b, 0, 0, 0)),
            pl.BlockSpec((_BB, _H, _W, _D), lambda b: (2 * b + 1, 0, 0, 0)),
            pl.BlockSpec((_D, 128), lambda b: (0, 0)),
            pl.BlockSpec((128, 128), lambda b: (0, 0)),
            pl.BlockSpec((1, 1, 1, _K), lambda b: (0, 0, 0, 0)),
        ],
        out_specs=[
            pl.BlockSpec((_BB, _H, _W, _K), lambda b: (2 * b, 0, 0, 0)),
            pl.BlockSpec((_BB, _H, _W, _K), lambda b: (2 * b + 1, 0, 0, 0)),
        ],
        out_shape=[
            jax.ShapeDtypeStruct((_B, _H, _W, _K), jnp.float32),
            jax.ShapeDtypeStruct((_B, _H, _W, _K), jnp.float32),
        ],
    )(tok2d, tok2d, wall, tmat, bias)
    bsel = (jnp.arange(_B) // _BB) % 2
    out = jnp.where(bsel[:, None, None, None] == 0, out1, out2)
    return out


# speedup vs baseline: 1.3512x; 1.3512x over previous
import jax
import jax.numpy as jnp
from jax.experimental import pallas as pl
from jax.experimental.pallas import tpu as pltpu

_B, _H, _W, _D = 64, 24, 24, 768
_K = 9
_BB = 8


def _conv_head_body(x_ref, w_ref, t_ref, b_ref, o_ref):
    x = x_ref[...].reshape(_BB * _H * _W, _D)
    z = jnp.dot(x, w_ref[...], preferred_element_type=jnp.float32)
    z = z.reshape(_BB, _H, _W, 128)
    shape = (_BB, _H, _W, 128)
    wio = jax.lax.broadcasted_iota(jnp.int32, shape, 2)
    cio = jax.lax.broadcasted_iota(jnp.int32, shape, 3)
    jg = (cio // _K) % 3
    pm = pltpu.roll(z, 1, axis=2)
    pp = pltpu.roll(z, _W - 1, axis=2)
    q = jnp.where(jg == 0, pm, jnp.where(jg == 1, z, pp))
    bad = ((jg == 0) & (wio == 0)) | ((jg == 2) & (wio == _W - 1))
    q = jnp.where(bad, 0.0, q)
    zrow = jnp.zeros((_BB, 1, _W, 128), jnp.float32)
    qp = jnp.concatenate([zrow, q, zrow], axis=1)
    ig = (cio // _K) // 3
    sl = [jax.lax.slice(qp, (0, i, 0, 0), (_BB, i + _H, _W, 128))
          for i in range(3)]
    s = jnp.where(ig == 0, sl[0], jnp.where(ig == 1, sl[1], sl[2]))
    s = s.reshape(_BB * _H * _W, 128)
    acc = jnp.dot(s, t_ref[...], preferred_element_type=jnp.float32)
    logits = jax.lax.slice(acc, (0, 0), (_BB * _H * _W, _K)) + b_ref[0, 0]
    m = jnp.max(logits, axis=-1, keepdims=True)
    e = jnp.exp(logits - m)
    den = jnp.sum(e, axis=-1, keepdims=True)
    o_ref[...] = (e / den).reshape(_BB, _H, _W, _K)


def kernel(tok2d, dw_w, pw_w, pw_b):
    dwt = dw_w.reshape(_D, 9)
    pwm = jnp.transpose(pw_w.reshape(_K, _D))
    wall = (dwt[:, :, None] * pwm[:, None, :]).reshape(_D, 81)
    wall = jnp.pad(wall, ((0, 0), (0, 47)))
    rows = jnp.arange(81)
    tmat = jnp.zeros((128, 128), jnp.float32).at[rows, rows % _K].set(1.0)
    bias = jnp.zeros((1, 1, 1, _K), jnp.float32).at[0, 0, 0, :].set(pw_b)
    out = pl.pallas_call(
        _conv_head_body,
        grid=(_B // _BB,),
        in_specs=[
            pl.BlockSpec((_BB, _H, _W, _D), lambda b: (b, 0, 0, 0)),
            pl.BlockSpec((_D, 128), lambda b: (0, 0)),
            pl.BlockSpec((128, 128), lambda b: (0, 0)),
            pl.BlockSpec((1, 1, 1, _K), lambda b: (0, 0, 0, 0)),
        ],
        out_specs=pl.BlockSpec((_BB, _H, _W, _K), lambda b: (b, 0, 0, 0)),
        out_shape=jax.ShapeDtypeStruct((_B, _H, _W, _K), jnp.float32),
    )(tok2d, wall, tmat, bias)
    return out


# speedup vs baseline: 1.3515x; 1.0002x over previous
import jax
import jax.numpy as jnp
from jax.experimental import pallas as pl
from jax.experimental.pallas import tpu as pltpu

_B, _H, _W, _D = 64, 24, 24, 768
_K = 9
_BB = 8


_CH = 8


def _conv_head_body(x_ref, w_ref, t_ref, b_ref, o_ref):
    for c0 in range(0, _BB, _CH):
        _conv_head_chunk(x_ref, w_ref, t_ref, b_ref, o_ref, c0)


def _conv_head_chunk(x_ref, w_ref, t_ref, b_ref, o_ref, c0):
    x = x_ref[c0:c0 + _CH].reshape(_CH * _H * _W, _D)
    z = jnp.dot(x, w_ref[...], preferred_element_type=jnp.float32)
    z = z.reshape(_CH, _H, _W, 128)
    shape = (_CH, _H, _W, 128)
    wio = jax.lax.broadcasted_iota(jnp.int32, shape, 2)
    cio = jax.lax.broadcasted_iota(jnp.int32, shape, 3)
    jg = (cio // _K) % 3
    pm = pltpu.roll(z, 1, axis=2)
    pp = pltpu.roll(z, _W - 1, axis=2)
    q = jnp.where(jg == 0, pm, jnp.where(jg == 1, z, pp))
    bad = ((jg == 0) & (wio == 0)) | ((jg == 2) & (wio == _W - 1))
    q = jnp.where(bad, 0.0, q)
    zrow = jnp.zeros((_CH, 1, _W, 128), jnp.float32)
    qp = jnp.concatenate([zrow, q, zrow], axis=1)
    ig = (cio // _K) // 3
    sl = [jax.lax.slice(qp, (0, i, 0, 0), (_CH, i + _H, _W, 128))
          for i in range(3)]
    s = jnp.where(ig == 0, sl[0], jnp.where(ig == 1, sl[1], sl[2]))
    s = s.reshape(_CH * _H * _W, 128)
    acc = jnp.dot(s, t_ref[...], preferred_element_type=jnp.float32)
    logits = jax.lax.slice(acc, (0, 0), (_CH * _H * _W, _K)) + b_ref[0, 0]
    m = jnp.max(logits, axis=-1, keepdims=True)
    e = jnp.exp(logits - m)
    den = jnp.sum(e, axis=-1, keepdims=True)
    o_ref[c0:c0 + _CH] = (e / den).reshape(_CH, _H, _W, _K)


def kernel(tok2d, dw_w, pw_w, pw_b):
    dwt = dw_w.reshape(_D, 9)
    pwm = jnp.transpose(pw_w.reshape(_K, _D))
    wall = (dwt[:, :, None] * pwm[:, None, :]).reshape(_D, 81)
    wall = jnp.pad(wall, ((0, 0), (0, 47)))
    rows = jnp.arange(81)
    tmat = jnp.zeros((128, 128), jnp.float32).at[rows, rows % _K].set(1.0)
    bias = jnp.zeros((1, 1, 1, _K), jnp.float32).at[0, 0, 0, :].set(pw_b)
    out = pl.pallas_call(
        _conv_head_body,
        grid=(_B // _BB,),
        in_specs=[
            pl.BlockSpec((_BB, _H, _W, _D), lambda b: (b, 0, 0, 0)),
            pl.BlockSpec((_D, 128), lambda b: (0, 0)),
            pl.BlockSpec((128, 128), lambda b: (0, 0)),
            pl.BlockSpec((1, 1, 1, _K), lambda b: (0, 0, 0, 0)),
        ],
        out_specs=pl.BlockSpec((_BB, _H, _W, _K), lambda b: (b, 0, 0, 0)),
        out_shape=jax.ShapeDtypeStruct((_B, _H, _W, _K), jnp.float32),
        compiler_params=pltpu.CompilerParams(
            dimension_semantics=("arbitrary",),
        ),
    )(tok2d, wall, tmat, bias)
    return out
